# Initial kernel scaffold; baseline (speedup 1.0000x reference)
import jax
import jax.numpy as jnp
from jax.experimental import pallas as pl


def kernel(feats_final, labels, sample_ids):
    raise NotImplementedError("write your pallas kernel here")


# speedup vs baseline: 3.7682x; 3.7682x over previous
import jax
import jax.numpy as jnp
from jax.experimental import pallas as pl

_B = 4096
_D = 128
_K = 5
_NEG = -1e9
_TR = 256
_NBLK = _B // _TR


def _normalize_kernel(x_ref, o_ref):
    x = x_ref[...]
    n = jnp.sqrt(jnp.sum(x * x, axis=-1, keepdims=True))
    o_ref[...] = x / jnp.maximum(n, 1e-8)


def _sim_tile(rows, cols):
    sim = jax.lax.dot_general(rows, cols, (((1,), (0,)), ((), ())),
                              preferred_element_type=jnp.float32)
    return jnp.clip(sim, -1.0 + 1e-8, 1.0 - 1e-8)


def _base_masks(labr_ref, labc_ref, mr_ref, mc_ref, r0):
    same = labr_ref[...] == labc_ref[...]
    m_r = mr_ref[...] > 0.5
    m_c = mc_ref[...] > 0.5
    riota = jax.lax.broadcasted_iota(jnp.int32, (_TR, _B), 0) + r0
    ciota = jax.lax.broadcasted_iota(jnp.int32, (_TR, _B), 1)
    diag = riota == ciota
    return same & m_r & m_c, diag


def _fifth_largest(masked):
    cur = masked
    for _ in range(_K - 1):
        mx = jnp.max(cur, axis=1, keepdims=True)
        cur = jnp.where(cur == mx, _NEG, cur)
    return jnp.max(cur, axis=1, keepdims=True)


def _stats_kernel(xn_ref, xnt_ref, labr_ref, labc_ref, mr_ref, mc_ref,
                  t_ref, keep_ref, ema_ref):
    i = pl.program_id(1)
    r0 = i * _TR
    sim = _sim_tile(xn_ref[0], xnt_ref[0])
    base, diag = _base_masks(labr_ref, labc_ref, mr_ref, mc_ref, r0)
    pos = sim > 0.0
    cand_m = base & (~diag) & pos
    masked = jnp.where(cand_m, sim, _NEG)
    cand = jnp.sum(cand_m.astype(jnp.float32), axis=1, keepdims=True)
    keep = (cand >= float(_K)).astype(jnp.float32)
    t5 = _fifth_largest(masked)
    intra = base & pos
    ssum = jnp.sum(jnp.where(intra, sim, 0.0), axis=1, keepdims=True)
    dcnt = jnp.sum(intra.astype(jnp.float32), axis=1, keepdims=True)
    deg = jnp.maximum(dcnt, 1.0)
    ema = 0.45 + 0.1 * jax.nn.sigmoid(ssum / deg)
    t_ref[0] = t5
    keep_ref[0] = keep
    ema_ref[0] = ema


def _cand_kernel(xn_ref, xnt_ref, labr_ref, labc_ref, mr_ref, mc_ref,
                 tr_ref, tc_ref, kr_ref, kc_ref, cand_ref):
    i = pl.program_id(0)
    r0 = i * _TR
    sim = _sim_tile(xn_ref[0], xnt_ref[0])
    base, diag = _base_masks(labr_ref, labc_ref, mr_ref, mc_ref, r0)
    kp_r = kr_ref[...] > 0.5
    kp_c = kc_ref[...] > 0.5
    tmin = jnp.minimum(tr_ref[...], tc_ref[...])
    mem = kp_r & kp_c & base & (~diag) & (sim >= tmin)
    cand_ref[...] = jnp.max(mem.astype(jnp.float32), axis=1, keepdims=True)


def _loss_kernel(xn_ref, xnt_ref, labr_ref, labc_ref, mr_ref, mc_ref,
                 tr_ref, tc_ref, kr_ref, kc_ref, er_ref, ec_ref,
                 num_ref, den_ref):
    i = pl.program_id(0)
    r0 = i * _TR

    @pl.when(i == 0)
    def _():
        num_ref[...] = jnp.zeros_like(num_ref)
        den_ref[...] = jnp.zeros_like(den_ref)

    base, diag = _base_masks(labr_ref, labc_ref, mr_ref, mc_ref, r0)
    allowed = base & (~diag)

    def layer_arrays(l):
        sim = _sim_tile(xn_ref[l], xnt_ref[l])
        tmin = jnp.minimum(tr_ref[l], tc_ref[l])
        kp_r = kr_ref[l] > 0.5
        kp_c = kc_ref[l] > 0.5
        mask = kp_r & kp_c & allowed & (sim >= tmin)
        a = jnp.where(mask, sim, 0.0)
        return mask, a

    lane = jax.lax.broadcasted_iota(jnp.int32, (1, 128), 1)
    num_acc = jnp.zeros((1, 128), jnp.float32)
    den_acc = jnp.zeros((1, 128), jnp.float32)
    mask_p, a_p = layer_arrays(0)
    for l in range(1, 4):
        mask_c, a_c = layer_arrays(l)
        w = er_ref[l - 1] * ec_ref[l - 1]
        meff = jnp.where(mask_p | mask_c, w, 0.0)
        d = a_c - a_p
        pnum = jnp.sum(d * d * meff)
        pden = jnp.sum(meff)
        sel = (lane == (l - 1)).astype(jnp.float32)
        num_acc = num_acc + pnum * sel
        den_acc = den_acc + pden * sel
        mask_p, a_p = mask_c, a_c
    num_ref[...] += num_acc
    den_ref[...] += den_acc


def kernel(feats_final, labels, sample_ids):
    del sample_ids
    feats = feats_final.astype(jnp.float32)
    labf_c = labels.astype(jnp.float32).reshape(1, _B)
    labf_r = labf_c.reshape(_B, 1)
    ones_c = jnp.ones((1, _B), jnp.float32)
    ones_r = jnp.ones((_B, 1), jnp.float32)

    xn = pl.pallas_call(
        _normalize_kernel,
        out_shape=jax.ShapeDtypeStruct((4, _B, _D), jnp.float32),
    )(feats)
    xnt = jnp.swapaxes(xn, 1, 2)

    def _rvec(nargs=None):
        return pl.BlockSpec((_TR, 1), lambda *a: (a[-1], 0))

    def _cvec():
        return pl.BlockSpec((1, _B), lambda *a: (0, 0))

    def stats(xn_in, xnt_in, m_r, m_c):
        nl = xn_in.shape[0]
        out = jax.ShapeDtypeStruct((nl, _B, 1), jnp.float32)
        ospec = pl.BlockSpec((1, _TR, 1), lambda l, i: (l, i, 0))
        return pl.pallas_call(
            _stats_kernel,
            grid=(nl, _NBLK),
            in_specs=[
                pl.BlockSpec((1, _TR, _D), lambda l, i: (l, i, 0)),
                pl.BlockSpec((1, _D, _B), lambda l, i: (l, 0, 0)),
                _rvec(), _cvec(), _rvec(), _cvec(),
            ],
            out_specs=[ospec] * 3,
            out_shape=[out, out, out],
        )(xn_in, xnt_in, labf_r, labf_c, m_r, m_c)

    xn_last = jax.lax.slice_in_dim(xn, 3, 4, axis=0)
    xnt_last = jax.lax.slice_in_dim(xnt, 3, 4, axis=0)
    t_last, keep_last, _ = stats(xn_last, xnt_last, ones_r, ones_c)
    tl_r = t_last.reshape(_B, 1)
    tl_c = t_last.reshape(1, _B)
    kl_r = keep_last.reshape(_B, 1)
    kl_c = keep_last.reshape(1, _B)

    cand_args = (xn_last, xnt_last, labf_r, labf_c, ones_r, ones_c,
                 tl_r, tl_c, kl_r, kl_c)
    m_r = pl.pallas_call(
        _cand_kernel,
        grid=(_NBLK,),
        in_specs=[
            pl.BlockSpec((1, _TR, _D), lambda i: (0, i, 0)),
            pl.BlockSpec((1, _D, _B), lambda i: (0, 0, 0)),
            _rvec(), _cvec(), _rvec(), _cvec(),
            _rvec(), _cvec(), _rvec(), _cvec(),
        ],
        out_specs=pl.BlockSpec((_TR, 1), lambda i: (i, 0)),
        out_shape=jax.ShapeDtypeStruct((_B, 1), jnp.float32),
    )(*cand_args)
    m_c = m_r.reshape(1, _B)

    t, keep, ema = stats(xn, xnt, m_r, m_c)
    t_c = jnp.swapaxes(t, 1, 2)
    keep_c = jnp.swapaxes(keep, 1, 2)
    ema_c = jnp.swapaxes(ema, 1, 2)

    def _rvec3():
        return pl.BlockSpec((4, _TR, 1), lambda i: (0, i, 0))

    def _cvec3():
        return pl.BlockSpec((4, 1, _B), lambda i: (0, 0, 0))

    loss_args = (xn, xnt, labf_r, labf_c, m_r, m_c,
                 t, t_c, keep, keep_c, ema, ema_c)
    num, den = pl.pallas_call(
        _loss_kernel,
        grid=(_NBLK,),
        in_specs=[
            pl.BlockSpec((4, _TR, _D), lambda i: (0, i, 0)),
            pl.BlockSpec((4, _D, _B), lambda i: (0, 0, 0)),
            _rvec(), _cvec(), _rvec(), _cvec(),
            _rvec3(), _cvec3(), _rvec3(), _cvec3(), _rvec3(), _cvec3(),
        ],
        out_specs=[pl.BlockSpec((1, 128), lambda i: (0, 0))] * 2,
        out_shape=[jax.ShapeDtypeStruct((1, 128), jnp.float32)] * 2,
    )(*loss_args)

    num3 = num[0, :3]
    den3 = den[0, :3]
    loss = jnp.sum(num3 / jnp.maximum(den3, 1e-8))
    raw = loss / 3.0
    gate = jnp.sum(m_r) >= 2.0
    raw = jnp.where(gate, raw, jnp.zeros(()))
    return (raw, 16.0 * raw)


# speedup vs baseline: 4.5793x; 1.2153x over previous
import jax
import jax.numpy as jnp
from jax.experimental import pallas as pl

_B = 4096
_D = 128
_K = 5
_NEG = -1e9
_TR = 256
_NBLK = _B // _TR


def _normalize_kernel(x_ref, o_ref):
    x = x_ref[...]
    n = jnp.sqrt(jnp.sum(x * x, axis=-1, keepdims=True))
    o_ref[...] = x / jnp.maximum(n, 1e-8)


def _sim_tile(rows, cols):
    return jax.lax.dot_general(rows, cols, (((1,), (0,)), ((), ())),
                               preferred_element_type=jnp.float32)


def _base_masks(labr_ref, labc_ref, mr_ref, mc_ref, r0):
    same = labr_ref[...] == labc_ref[...]
    m_r = mr_ref[...] > 0.5
    m_c = mc_ref[...] > 0.5
    riota = jax.lax.broadcasted_iota(jnp.int32, (_TR, _B), 0) + r0
    ciota = jax.lax.broadcasted_iota(jnp.int32, (_TR, _B), 1)
    diag = riota == ciota
    return same & m_r & m_c, diag


def _fifth_largest(masked):
    cur = masked
    for _ in range(_K - 1):
        mx = jnp.max(cur, axis=1, keepdims=True)
        cur = jnp.where(cur == mx, _NEG, cur)
    return jnp.max(cur, axis=1, keepdims=True)


def _stats_kernel(xn_ref, xnt_ref, labr_ref, labc_ref, mr_ref, mc_ref,
                  t_ref, keep_ref, ema_ref):
    i = pl.program_id(0)
    r0 = i * _TR
    base, diag = _base_masks(labr_ref, labc_ref, mr_ref, mc_ref, r0)
    nondiag_base = base & (~diag)
    m_r = mr_ref[...]
    for l in range(xn_ref.shape[0]):
        sim = _sim_tile(xn_ref[l], xnt_ref[l])
        cand_m = nondiag_base & (sim > 0.0)
        cand = jnp.sum(cand_m.astype(jnp.float32), axis=1, keepdims=True)
        keep = (cand >= float(_K)).astype(jnp.float32)
        t5 = _fifth_largest(jnp.where(cand_m, sim, _NEG))
        ssum = jnp.sum(jnp.where(cand_m, sim, 0.0), axis=1, keepdims=True)
        ssum = ssum + (1.0 - 1e-8) * m_r
        deg = jnp.maximum(cand + m_r, 1.0)
        ema = 0.45 + 0.1 * jax.nn.sigmoid(ssum / deg)
        t_ref[l] = t5
        keep_ref[l] = keep
        ema_ref[l] = ema


def _cand_kernel(xn_ref, xnt_ref, labr_ref, labc_ref, mr_ref, mc_ref,
                 tr_ref, tc_ref, kr_ref, kc_ref, cand_ref):
    i = pl.program_id(0)
    r0 = i * _TR
    sim = _sim_tile(xn_ref[0], xnt_ref[0])
    base, diag = _base_masks(labr_ref, labc_ref, mr_ref, mc_ref, r0)
    kp_r = kr_ref[...] > 0.5
    kp_c = kc_ref[...] > 0.5
    tmin = jnp.minimum(tr_ref[...], tc_ref[...])
    mem = kp_r & kp_c & base & (~diag) & (sim >= tmin)
    cand_ref[...] = jnp.max(mem.astype(jnp.float32), axis=1, keepdims=True)


def _loss_kernel(xn_ref, xnt_ref, labr_ref, labc_ref, mr_ref, mc_ref,
                 tr_ref, tc_ref, kr_ref, kc_ref, er_ref, ec_ref,
                 num_ref, den_ref):
    i = pl.program_id(0)
    r0 = i * _TR

    @pl.when(i == 0)
    def _():
        num_ref[...] = jnp.zeros_like(num_ref)
        den_ref[...] = jnp.zeros_like(den_ref)

    base, diag = _base_masks(labr_ref, labc_ref, mr_ref, mc_ref, r0)
    allowed = base & (~diag)

    def layer_arrays(l):
        sim = _sim_tile(xn_ref[l], xnt_ref[l])
        tmin = jnp.minimum(tr_ref[l], tc_ref[l])
        kp_r = kr_ref[l] > 0.5
        kp_c = kc_ref[l] > 0.5
        mask = kp_r & kp_c & allowed & (sim >= tmin)
        a = jnp.where(mask, sim, 0.0)
        return mask, a

    lane = jax.lax.broadcasted_iota(jnp.int32, (1, 128), 1)
    num_acc = jnp.zeros((1, 128), jnp.float32)
    den_acc = jnp.zeros((1, 128), jnp.float32)
    mask_p, a_p = layer_arrays(0)
    for l in range(1, 4):
        mask_c, a_c = layer_arrays(l)
        w = er_ref[l - 1] * ec_ref[l - 1]
        meff = jnp.where(mask_p | mask_c, w, 0.0)
        d = a_c - a_p
        pnum = jnp.sum(d * d * meff)
        pden = jnp.sum(meff)
        sel = (lane == (l - 1)).astype(jnp.float32)
        num_acc = num_acc + pnum * sel
        den_acc = den_acc + pden * sel
        mask_p, a_p = mask_c, a_c
    num_ref[...] += num_acc
    den_ref[...] += den_acc


def kernel(feats_final, labels, sample_ids):
    del sample_ids
    feats = feats_final.astype(jnp.float32)
    labf_c = labels.astype(jnp.float32).reshape(1, _B)
    labf_r = labf_c.reshape(_B, 1)
    ones_c = jnp.ones((1, _B), jnp.float32)
    ones_r = jnp.ones((_B, 1), jnp.float32)

    xn = pl.pallas_call(
        _normalize_kernel,
        out_shape=jax.ShapeDtypeStruct((4, _B, _D), jnp.float32),
    )(feats)
    xnt = jnp.swapaxes(xn, 1, 2)

    def _rvec(nargs=None):
        return pl.BlockSpec((_TR, 1), lambda *a: (a[-1], 0))

    def _cvec():
        return pl.BlockSpec((1, _B), lambda *a: (0, 0))

    def stats(xn_in, xnt_in, m_r, m_c):
        nl = xn_in.shape[0]
        out = jax.ShapeDtypeStruct((nl, _B, 1), jnp.float32)
        ospec = pl.BlockSpec((nl, _TR, 1), lambda i: (0, i, 0))
        return pl.pallas_call(
            _stats_kernel,
            grid=(_NBLK,),
            in_specs=[
                pl.BlockSpec((nl, _TR, _D), lambda i: (0, i, 0)),
                pl.BlockSpec((nl, _D, _B), lambda i: (0, 0, 0)),
                _rvec(), _cvec(), _rvec(), _cvec(),
            ],
            out_specs=[ospec] * 3,
            out_shape=[out, out, out],
        )(xn_in, xnt_in, labf_r, labf_c, m_r, m_c)

    xn_last = jax.lax.slice_in_dim(xn, 3, 4, axis=0)
    xnt_last = jax.lax.slice_in_dim(xnt, 3, 4, axis=0)
    t_last, keep_last, _ = stats(xn_last, xnt_last, ones_r, ones_c)
    tl_r = t_last.reshape(_B, 1)
    tl_c = t_last.reshape(1, _B)
    kl_r = keep_last.reshape(_B, 1)
    kl_c = keep_last.reshape(1, _B)

    cand_args = (xn_last, xnt_last, labf_r, labf_c, ones_r, ones_c,
                 tl_r, tl_c, kl_r, kl_c)
    m_r = pl.pallas_call(
        _cand_kernel,
        grid=(_NBLK,),
        in_specs=[
            pl.BlockSpec((1, _TR, _D), lambda i: (0, i, 0)),
            pl.BlockSpec((1, _D, _B), lambda i: (0, 0, 0)),
            _rvec(), _cvec(), _rvec(), _cvec(),
            _rvec(), _cvec(), _rvec(), _cvec(),
        ],
        out_specs=pl.BlockSpec((_TR, 1), lambda i: (i, 0)),
        out_shape=jax.ShapeDtypeStruct((_B, 1), jnp.float32),
    )(*cand_args)
    m_c = m_r.reshape(1, _B)

    t, keep, ema = stats(xn, xnt, m_r, m_c)
    t_c = jnp.swapaxes(t, 1, 2)
    keep_c = jnp.swapaxes(keep, 1, 2)
    ema_c = jnp.swapaxes(ema, 1, 2)

    def _rvec3():
        return pl.BlockSpec((4, _TR, 1), lambda i: (0, i, 0))

    def _cvec3():
        return pl.BlockSpec((4, 1, _B), lambda i: (0, 0, 0))

    loss_args = (xn, xnt, labf_r, labf_c, m_r, m_c,
                 t, t_c, keep, keep_c, ema, ema_c)
    num, den = pl.pallas_call(
        _loss_kernel,
        grid=(_NBLK,),
        in_specs=[
            pl.BlockSpec((4, _TR, _D), lambda i: (0, i, 0)),
            pl.BlockSpec((4, _D, _B), lambda i: (0, 0, 0)),
            _rvec(), _cvec(), _rvec(), _cvec(),
            _rvec3(), _cvec3(), _rvec3(), _cvec3(), _rvec3(), _cvec3(),
        ],
        out_specs=[pl.BlockSpec((1, 128), lambda i: (0, 0))] * 2,
        out_shape=[jax.ShapeDtypeStruct((1, 128), jnp.float32)] * 2,
    )(*loss_args)

    num3 = num[0, :3]
    den3 = den[0, :3]
    loss = jnp.sum(num3 / jnp.maximum(den3, 1e-8))
    raw = loss / 3.0
    gate = jnp.sum(m_r) >= 2.0
    raw = jnp.where(gate, raw, jnp.zeros(()))
    return (raw, 16.0 * raw)


# speedup vs baseline: 6.5576x; 1.4320x over previous
import jax
import jax.numpy as jnp
from jax.experimental import pallas as pl

_B = 4096
_D = 128
_K = 5
_NEG = -1e9
_BIG = 1e9
_TR = 256
_NBLK = _B // _TR


def _normalize_kernel(x_ref, o_ref):
    x = x_ref[...]
    n = jnp.sqrt(jnp.sum(x * x, axis=-1, keepdims=True))
    o_ref[...] = x / jnp.maximum(n, 1e-8)


def _sim_tile(rows, cols):
    return jax.lax.dot_general(rows, cols, (((1,), (0,)), ((), ())),
                               preferred_element_type=jnp.float32)


def _amask(labr, labc, r0, m_r=None, m_c=None):
    ok = labr == labc
    if m_r is not None:
        ok = ok & (m_r > 0.5) & (m_c > 0.5)
    riota = jax.lax.broadcasted_iota(jnp.int32, (_TR, _B), 0) + r0
    ciota = jax.lax.broadcasted_iota(jnp.int32, (_TR, _B), 1)
    ok = ok & (riota != ciota)
    return jnp.where(ok, 0.0, _NEG)


def _fifth_largest(masked):
    cur = masked
    for _ in range(_K - 1):
        mx = jnp.max(cur, axis=1, keepdims=True)
        cur = jnp.where(cur == mx, _NEG, cur)
    return jnp.max(cur, axis=1, keepdims=True)


def _make_stats_kernel(with_ema):
    def _stats_kernel(xn_ref, xnt_ref, labr_ref, labc_ref, mr_ref, mc_ref,
                      u_ref, *rest):
        i = pl.program_id(0)
        m_r = mr_ref[...]
        amask = _amask(labr_ref[...], labc_ref[...], i * _TR,
                       m_r, mc_ref[...])
        for l in range(xn_ref.shape[0]):
            sim = _sim_tile(xn_ref[l], xnt_ref[l])
            masked = sim + amask
            t5 = _fifth_largest(masked)
            u_ref[l] = jnp.where(t5 > 0.0, t5, _BIG)
            if with_ema:
                ema_ref = rest[0]
                pos = jnp.maximum(masked, 0.0)
                ind = (masked > 0.0).astype(jnp.float32)
                ssum = jnp.sum(pos, axis=1, keepdims=True)
                cand = jnp.sum(ind, axis=1, keepdims=True)
                ssum = ssum + (1.0 - 1e-8) * m_r
                deg = jnp.maximum(cand + m_r, 1.0)
                ema_ref[l] = 0.45 + 0.1 * jax.nn.sigmoid(ssum / deg)
    return _stats_kernel


def _cand_kernel(xn_ref, xnt_ref, labr_ref, labc_ref, ur_ref, uc_ref,
                 cand_ref):
    i = pl.program_id(0)
    amask = _amask(labr_ref[...], labc_ref[...], i * _TR)
    sim = _sim_tile(xn_ref[0], xnt_ref[0])
    thr = jnp.minimum(ur_ref[...], uc_ref[...])
    z = jnp.max((sim + amask) - thr, axis=1, keepdims=True)
    cand_ref[...] = (z >= 0.0).astype(jnp.float32)


def _loss_kernel(xn_ref, xnt_ref, labr_ref, labc_ref, mr_ref, mc_ref,
                 ur_ref, uc_ref, er_ref, ecol_ref, num_ref, den_ref):
    i = pl.program_id(0)

    @pl.when(i == 0)
    def _():
        num_ref[...] = jnp.zeros_like(num_ref)
        den_ref[...] = jnp.zeros_like(den_ref)

    amask = _amask(labr_ref[...], labc_ref[...], i * _TR,
                   mr_ref[...], mc_ref[...])

    def layer_arrays(l):
        sim = _sim_tile(xn_ref[l], xnt_ref[l])
        s = sim + amask
        thr = jnp.minimum(ur_ref[l], uc_ref[l])
        ind = (s >= thr).astype(jnp.float32)
        return ind, s * ind

    def matvec(m, v):
        return jax.lax.dot_general(m, v, (((1,), (0,)), ((), ())),
                                   preferred_element_type=jnp.float32)

    lane = jax.lax.broadcasted_iota(jnp.int32, (1, 128), 1)
    num_acc = jnp.zeros((1, 128), jnp.float32)
    den_acc = jnp.zeros((1, 128), jnp.float32)
    ind_p, a_p = layer_arrays(0)
    for l in range(1, 4):
        ind_c, a_c = layer_arrays(l)
        ind_pc = jnp.maximum(ind_p, ind_c)
        d = a_c - a_p
        q = d * d * ind_pc
        ecol = ecol_ref[l - 1]
        er = er_ref[l - 1]
        pnum = jnp.sum(er * matvec(q, ecol))
        pden = jnp.sum(er * matvec(ind_pc, ecol))
        sel = (lane == (l - 1)).astype(jnp.float32)
        num_acc = num_acc + pnum * sel
        den_acc = den_acc + pden * sel
        ind_p, a_p = ind_c, a_c
    num_ref[...] += num_acc
    den_ref[...] += den_acc


def kernel(feats_final, labels, sample_ids):
    del sample_ids
    feats = feats_final.astype(jnp.float32)
    labf_c = labels.astype(jnp.float32).reshape(1, _B)
    labf_r = labf_c.reshape(_B, 1)
    ones_c = jnp.ones((1, _B), jnp.float32)
    ones_r = jnp.ones((_B, 1), jnp.float32)

    xn = pl.pallas_call(
        _normalize_kernel,
        out_shape=jax.ShapeDtypeStruct((4, _B, _D), jnp.float32),
    )(feats)
    xnt = jnp.swapaxes(xn, 1, 2)

    def _rvec():
        return pl.BlockSpec((_TR, 1), lambda i: (i, 0))

    def _cvec():
        return pl.BlockSpec((1, _B), lambda i: (0, 0))

    def stats(xn_in, xnt_in, m_r, m_c, with_ema):
        nl = xn_in.shape[0]
        out = jax.ShapeDtypeStruct((nl, _B, 1), jnp.float32)
        ospec = pl.BlockSpec((nl, _TR, 1), lambda i: (0, i, 0))
        n_out = 2 if with_ema else 1
        return pl.pallas_call(
            _make_stats_kernel(with_ema),
            grid=(_NBLK,),
            in_specs=[
                pl.BlockSpec((nl, _TR, _D), lambda i: (0, i, 0)),
                pl.BlockSpec((nl, _D, _B), lambda i: (0, 0, 0)),
                _rvec(), _cvec(), _rvec(), _cvec(),
            ],
            out_specs=[ospec] * n_out,
            out_shape=[out] * n_out,
        )(xn_in, xnt_in, labf_r, labf_c, m_r, m_c)

    xn_last = jax.lax.slice_in_dim(xn, 3, 4, axis=0)
    xnt_last = jax.lax.slice_in_dim(xnt, 3, 4, axis=0)
    (u_last,) = stats(xn_last, xnt_last, ones_r, ones_c, False)
    ul_r = u_last.reshape(_B, 1)
    ul_c = u_last.reshape(1, _B)

    m_r = pl.pallas_call(
        _cand_kernel,
        grid=(_NBLK,),
        in_specs=[
            pl.BlockSpec((1, _TR, _D), lambda i: (0, i, 0)),
            pl.BlockSpec((1, _D, _B), lambda i: (0, 0, 0)),
            _rvec(), _cvec(), _rvec(), _cvec(),
        ],
        out_specs=pl.BlockSpec((_TR, 1), lambda i: (i, 0)),
        out_shape=jax.ShapeDtypeStruct((_B, 1), jnp.float32),
    )(xn_last, xnt_last, labf_r, labf_c, ul_r, ul_c)
    m_c = m_r.reshape(1, _B)

    u, ema = stats(xn, xnt, m_r, m_c, True)
    u_c = jnp.swapaxes(u, 1, 2)

    loss_args = (xn, xnt, labf_r, labf_c, m_r, m_c, u, u_c, ema, ema)
    num, den = pl.pallas_call(
        _loss_kernel,
        grid=(_NBLK,),
        in_specs=[
            pl.BlockSpec((4, _TR, _D), lambda i: (0, i, 0)),
            pl.BlockSpec((4, _D, _B), lambda i: (0, 0, 0)),
            _rvec(), _cvec(), _rvec(), _cvec(),
            pl.BlockSpec((4, _TR, 1), lambda i: (0, i, 0)),
            pl.BlockSpec((4, 1, _B), lambda i: (0, 0, 0)),
            pl.BlockSpec((4, _TR, 1), lambda i: (0, i, 0)),
            pl.BlockSpec((4, _B, 1), lambda i: (0, 0, 0)),
        ],
        out_specs=[pl.BlockSpec((1, 128), lambda i: (0, 0))] * 2,
        out_shape=[jax.ShapeDtypeStruct((1, 128), jnp.float32)] * 2,
    )(*loss_args)

    num3 = num[0, :3]
    den3 = den[0, :3]
    loss = jnp.sum(num3 / jnp.maximum(den3, 1e-8))
    raw = loss / 3.0
    gate = jnp.sum(m_r) >= 2.0
    raw = jnp.where(gate, raw, jnp.zeros(()))
    return (raw, 16.0 * raw)


# speedup vs baseline: 6.9586x; 1.0611x over previous
import jax
import jax.numpy as jnp
from jax.experimental import pallas as pl

_B = 4096
_D = 128
_K = 5
_NEG = -1e9
_BIG = 1e9
_TR = 256
_NBLK = _B // _TR


def _normalize_kernel(x_ref, o_ref):
    x = x_ref[...]
    n = jnp.sqrt(jnp.sum(x * x, axis=-1, keepdims=True))
    o_ref[...] = x / jnp.maximum(n, 1e-8)


def _sim_tile(rows, cols):
    return jax.lax.dot_general(rows, cols, (((1,), (0,)), ((), ())),
                               preferred_element_type=jnp.float32)


def _amask(labr, labc, idxr, idxc, madd_r=None, madd_c=None):
    a = jnp.where(labr == labc, 0.0, _NEG)
    d = jnp.where(idxr == idxc, _NEG, 0.0)
    amask = a + d
    if madd_r is not None:
        amask = amask + (madd_r + madd_c)
    return amask


def _fifth_largest(masked):
    cur = masked
    for _ in range(_K - 1):
        mx = jnp.max(cur, axis=1, keepdims=True)
        cur = jnp.where(cur == mx, _NEG, cur)
    return jnp.max(cur, axis=1, keepdims=True)


def _make_stats_kernel(with_ema):
    def _stats_kernel(xn_ref, xnt_ref, labr_ref, labc_ref, idxr_ref, idxc_ref,
                      mr_ref, mc_ref, u_ref, *rest):
        nl = xn_ref.shape[0]
        m_r = mr_ref[...]
        if with_ema:
            madd_r = (m_r - 1.0) * _BIG
            madd_c = (mc_ref[...] - 1.0) * _BIG
        else:
            madd_r = madd_c = None
        amask = _amask(labr_ref[...], labc_ref[...], idxr_ref[...],
                       idxc_ref[...], madd_r, madd_c)
        for l in range(nl):
            sim = _sim_tile(xn_ref[l], xnt_ref[l])
            masked = sim + amask
            t5 = _fifth_largest(masked)
            u_ref[l] = jnp.where(t5 > 0.0, t5, _BIG)
            if with_ema and l < 3:
                ema_ref = rest[0]
                pos = jnp.maximum(masked, 0.0)
                ind = (masked > 0.0).astype(jnp.float32)
                ssum = jnp.sum(pos, axis=1, keepdims=True)
                cand = jnp.sum(ind, axis=1, keepdims=True)
                ssum = ssum + (1.0 - 1e-8) * m_r
                deg = jnp.maximum(cand + m_r, 1.0)
                ema_ref[l] = 0.45 + 0.1 * jax.nn.sigmoid(ssum / deg)
    return _stats_kernel


def _cand_kernel(xn_ref, xnt_ref, labr_ref, labc_ref, idxr_ref, idxc_ref,
                 ur_ref, uc_ref, cand_ref):
    amask = _amask(labr_ref[...], labc_ref[...], idxr_ref[...], idxc_ref[...])
    sim = _sim_tile(xn_ref[0], xnt_ref[0])
    thr = jnp.minimum(ur_ref[...], uc_ref[...])
    z = jnp.max((sim + amask) - thr, axis=1, keepdims=True)
    cand_ref[...] = (z >= 0.0).astype(jnp.float32)


def _loss_kernel(xn_ref, xnt_ref, labr_ref, labc_ref, idxr_ref, idxc_ref,
                 mr_ref, mc_ref, ur_ref, uc_ref, er_ref, ecol_ref,
                 num_ref, den_ref):
    i = pl.program_id(0)

    @pl.when(i == 0)
    def _():
        num_ref[...] = jnp.zeros_like(num_ref)
        den_ref[...] = jnp.zeros_like(den_ref)

    madd_r = (mr_ref[...] - 1.0) * _BIG
    madd_c = (mc_ref[...] - 1.0) * _BIG
    amask = _amask(labr_ref[...], labc_ref[...], idxr_ref[...], idxc_ref[...],
                   madd_r, madd_c)

    def layer_arrays(l):
        sim = _sim_tile(xn_ref[l], xnt_ref[l])
        s = sim + amask
        thr = jnp.minimum(ur_ref[l], uc_ref[l])
        ind = (s >= thr).astype(jnp.float32)
        return ind, s * ind

    def matvec(m, v):
        return jax.lax.dot_general(m, v, (((1,), (0,)), ((), ())),
                                   preferred_element_type=jnp.float32)

    lane = jax.lax.broadcasted_iota(jnp.int32, (1, 128), 1)
    num_acc = jnp.zeros((1, 128), jnp.float32)
    den_acc = jnp.zeros((1, 128), jnp.float32)
    ind_p, a_p = layer_arrays(0)
    for l in range(1, 4):
        ind_c, a_c = layer_arrays(l)
        ind_pc = jnp.maximum(ind_p, ind_c)
        d = a_c - a_p
        q = d * d * ind_pc
        ecol = ecol_ref[l - 1]
        er = er_ref[l - 1]
        pnum = jnp.sum(er * matvec(q, ecol))
        pden = jnp.sum(er * matvec(ind_pc, ecol))
        sel = (lane == (l - 1)).astype(jnp.float32)
        num_acc = num_acc + pnum * sel
        den_acc = den_acc + pden * sel
        ind_p, a_p = ind_c, a_c
    num_ref[...] += num_acc
    den_ref[...] += den_acc


def kernel(feats_final, labels, sample_ids):
    del sample_ids
    feats = feats_final.astype(jnp.float32)
    labf_c = labels.astype(jnp.float32).reshape(1, _B)
    labf_r = labf_c.reshape(_B, 1)
    idxf_c = jnp.arange(_B, dtype=jnp.float32).reshape(1, _B)
    idxf_r = idxf_c.reshape(_B, 1)
    ones_c = jnp.ones((1, _B), jnp.float32)
    ones_r = jnp.ones((_B, 1), jnp.float32)

    xn = pl.pallas_call(
        _normalize_kernel,
        out_shape=jax.ShapeDtypeStruct((4, _B, _D), jnp.float32),
    )(feats)
    xnt = jnp.swapaxes(xn, 1, 2)

    def _rvec():
        return pl.BlockSpec((_TR, 1), lambda i: (i, 0))

    def _cvec():
        return pl.BlockSpec((1, _B), lambda i: (0, 0))

    def stats(xn_in, xnt_in, m_r, m_c, with_ema):
        nl = xn_in.shape[0]
        out = jax.ShapeDtypeStruct((nl, _B, 1), jnp.float32)
        ospec = pl.BlockSpec((nl, _TR, 1), lambda i: (0, i, 0))
        n_out = 2 if with_ema else 1
        return pl.pallas_call(
            _make_stats_kernel(with_ema),
            grid=(_NBLK,),
            in_specs=[
                pl.BlockSpec((nl, _TR, _D), lambda i: (0, i, 0)),
                pl.BlockSpec((nl, _D, _B), lambda i: (0, 0, 0)),
                _rvec(), _cvec(), _rvec(), _cvec(), _rvec(), _cvec(),
            ],
            out_specs=[ospec] * n_out,
            out_shape=[out] * n_out,
        )(xn_in, xnt_in, labf_r, labf_c, idxf_r, idxf_c, m_r, m_c)

    xn_last = jax.lax.slice_in_dim(xn, 3, 4, axis=0)
    xnt_last = jax.lax.slice_in_dim(xnt, 3, 4, axis=0)
    (u_last,) = stats(xn_last, xnt_last, ones_r, ones_c, False)
    ul_r = u_last.reshape(_B, 1)
    ul_c = u_last.reshape(1, _B)

    m_r = pl.pallas_call(
        _cand_kernel,
        grid=(_NBLK,),
        in_specs=[
            pl.BlockSpec((1, _TR, _D), lambda i: (0, i, 0)),
            pl.BlockSpec((1, _D, _B), lambda i: (0, 0, 0)),
            _rvec(), _cvec(), _rvec(), _cvec(), _rvec(), _cvec(),
        ],
        out_specs=pl.BlockSpec((_TR, 1), lambda i: (i, 0)),
        out_shape=jax.ShapeDtypeStruct((_B, 1), jnp.float32),
    )(xn_last, xnt_last, labf_r, labf_c, idxf_r, idxf_c, ul_r, ul_c)
    m_c = m_r.reshape(1, _B)

    u, ema = stats(xn, xnt, m_r, m_c, True)
    u_c = jnp.swapaxes(u, 1, 2)

    loss_args = (xn, xnt, labf_r, labf_c, idxf_r, idxf_c, m_r, m_c,
                 u, u_c, ema, ema)
    num, den = pl.pallas_call(
        _loss_kernel,
        grid=(_NBLK,),
        in_specs=[
            pl.BlockSpec((4, _TR, _D), lambda i: (0, i, 0)),
            pl.BlockSpec((4, _D, _B), lambda i: (0, 0, 0)),
            _rvec(), _cvec(), _rvec(), _cvec(), _rvec(), _cvec(),
            pl.BlockSpec((4, _TR, 1), lambda i: (0, i, 0)),
            pl.BlockSpec((4, 1, _B), lambda i: (0, 0, 0)),
            pl.BlockSpec((4, _TR, 1), lambda i: (0, i, 0)),
            pl.BlockSpec((4, _B, 1), lambda i: (0, 0, 0)),
        ],
        out_specs=[pl.BlockSpec((1, 128), lambda i: (0, 0))] * 2,
        out_shape=[jax.ShapeDtypeStruct((1, 128), jnp.float32)] * 2,
    )(*loss_args)

    num3 = num[0, :3]
    den3 = den[0, :3]
    loss = jnp.sum(num3 / jnp.maximum(den3, 1e-8))
    raw = loss / 3.0
    gate = jnp.sum(m_r) >= 2.0
    raw = jnp.where(gate, raw, jnp.zeros(()))
    return (raw, 16.0 * raw)
